# Initial kernel scaffold; baseline (speedup 1.0000x reference)
import functools

import jax
import jax.numpy as jnp
import numpy as np
from jax import lax
from jax.experimental import pallas as pl
from jax.experimental.pallas import tpu as pltpu
from jax.experimental.pallas import tpu_sc as plsc

N = 10000
E = 320000
D = 128
H = 8
C = 16
DE = 16
DFF = 1024

NC = 2
NS = 16
L = 16

EPT_RAW = E // NS
BC = 128
NCHUNK = (EPT_RAW + BC - 1) // BC
EPT = NCHUNK * BC
EP = EPT * NS
ROWS_T = N // NS

EPW_RAW = E // (NC * NS)
NCHUNK2 = (EPW_RAW + BC - 1) // BC
EPW = NCHUNK2 * BC
EP2 = EPW * NC * NS


def _iota16():
    return lax.broadcasted_iota(jnp.int32, (L,), 0)


def _sc_mesh():
    return plsc.VectorSubcoreMesh(
        core_axis_name="c", subcore_axis_name="s", num_cores=NC,
        num_subcores=NS)


def _attn_body(qg_hbm, kv_hbm, src_hbm, dst_hbm, ea_hbm,
               outv_hbm, w2_hbm, den_hbm,
               src_i, dst_i, src_g, dst_g, kv_rows, qg_rows, ea_c,
               ex_c, msg_c, w2_c, den_c, zbuf, zden, sem1, sem2,
               sh_outv, sh_w2, sh_den):
    c = lax.axis_index("c")
    s = lax.axis_index("s")
    zero16 = jnp.zeros((L,), jnp.float32)

    def _z64(i, _):
        for jj in range(4):
            zbuf[i, pl.ds(jj * L, L)] = zero16
        return 0
    lax.fori_loop(0, 125, _z64, 0)

    def _z16(i, _):
        zden[i, :] = zero16
        return 0
    lax.fori_loop(0, 125, _z16, 0)

    for t in range(5):
        r0 = s * ROWS_T + t * 125
        pltpu.sync_copy(zbuf, sh_outv.at[pl.ds(r0, 125)])
        pltpu.sync_copy(zbuf, sh_w2.at[pl.ds(r0, 125)])
        pltpu.sync_copy(zden, sh_den.at[pl.ds(r0, 125)])
    plsc.subcore_barrier()

    coff = c * N
    iota = _iota16()

    def _chunk(ch, _):
        base = s * EPT + ch * BC
        pltpu.sync_copy(src_hbm.at[pl.ds(base, BC)], src_i)
        pltpu.sync_copy(dst_hbm.at[pl.ds(base, BC)], dst_i)
        pltpu.sync_copy(ea_hbm.at[pl.ds(base, BC)], ea_c)
        for g in range(BC // L):
            sl = pl.ds(g * L, L)
            src_g[sl] = src_i[sl] + coff
            dst_g[sl] = dst_i[sl] + coff
        pltpu.async_copy(kv_hbm.at[src_g], kv_rows, sem1).wait()
        pltpu.async_copy(qg_hbm.at[dst_g], qg_rows, sem2).wait()

        def _group(g, _):
            rows = g * L + iota
            acc = [zero16, zero16, zero16, zero16]
            for f in range(64):
                colf = jnp.full((L,), f, jnp.int32)
                qv = plsc.load_gather(qg_rows, [rows, colf])
                kv = plsc.load_gather(kv_rows, [rows, colf])
                acc[f >> 4] = acc[f >> 4] + qv * kv
            for j in range(DE):
                colj = jnp.full((L,), j, jnp.int32)
                eaj = plsc.load_gather(ea_c, [rows, colj])
                for h in range(4):
                    colg = jnp.full((L,), 64 + h * L + j, jnp.int32)
                    gv = plsc.load_gather(qg_rows, [rows, colg])
                    acc[h] = acc[h] + gv * eaj
            pos = ch * BC + g * L + iota
            valid = pos < EPT_RAW
            for h in range(4):
                al = jnp.minimum(acc[h] * 0.25, 60.0)
                exv = jnp.where(valid, jnp.exp(al), 0.0)
                ex_c[h, pl.ds(g * L, L)] = exv
            return 0
        lax.fori_loop(0, BC // L, _group, 0)

        def _edge(b, _):
            ea_b = ea_c[b, :]
            ridx = jnp.minimum(iota, 3)
            colb = jnp.full((L,), 0, jnp.int32) + b
            g1 = plsc.load_gather(ex_c, [ridx, colb])
            den_c[b, :] = jnp.where(iota < 4, g1, 0.0)
            for h in range(4):
                exb = plsc.load_gather(ex_c, [jnp.full((L,), h, jnp.int32),
                                              colb])
                v_b = kv_rows[b, pl.ds(64 + h * L, L)]
                msg_c[b, pl.ds(h * L, L)] = exb * v_b
                w2_c[b, pl.ds(h * L, L)] = exb * ea_b
            return 0
        lax.fori_loop(0, BC, _edge, 0)

        pltpu.sync_copy(msg_c, sh_outv.at[dst_i], add=True)
        pltpu.sync_copy(w2_c, sh_w2.at[dst_i], add=True)
        pltpu.sync_copy(den_c, sh_den.at[dst_i], add=True)
        return 0

    lax.fori_loop(0, NCHUNK, _chunk, 0)
    plsc.subcore_barrier()

    r0 = s * ROWS_T
    o0 = c * N + s * ROWS_T
    pltpu.sync_copy(sh_outv.at[pl.ds(r0, ROWS_T)], outv_hbm.at[pl.ds(o0, ROWS_T)])
    pltpu.sync_copy(sh_w2.at[pl.ds(r0, ROWS_T)], w2_hbm.at[pl.ds(o0, ROWS_T)])
    pltpu.sync_copy(sh_den.at[pl.ds(r0, ROWS_T)], den_hbm.at[pl.ds(o0, ROWS_T)])


@jax.jit
def _sc_attention(qg2, kv2, srcp, dstp, eap):
    f32 = jnp.float32
    kern = pl.kernel(
        _attn_body,
        out_type=[
            jax.ShapeDtypeStruct((NC * N, 64), f32),
            jax.ShapeDtypeStruct((NC * N, 64), f32),
            jax.ShapeDtypeStruct((NC * N, L), f32),
        ],
        mesh=_sc_mesh(),
        scratch_types=[
            pltpu.VMEM((BC,), jnp.int32),
            pltpu.VMEM((BC,), jnp.int32),
            pltpu.VMEM((BC,), jnp.int32),
            pltpu.VMEM((BC,), jnp.int32),
            pltpu.VMEM((BC, 128), f32),
            pltpu.VMEM((BC, 128), f32),
            pltpu.VMEM((BC, DE), f32),
            pltpu.VMEM((4, BC), f32),
            pltpu.VMEM((BC, 64), f32),
            pltpu.VMEM((BC, 64), f32),
            pltpu.VMEM((BC, L), f32),
            pltpu.VMEM((125, 64), f32),
            pltpu.VMEM((125, L), f32),
            pltpu.SemaphoreType.DMA,
            pltpu.SemaphoreType.DMA,
            pltpu.VMEM_SHARED((N, 64), f32),
            pltpu.VMEM_SHARED((N, 64), f32),
            pltpu.VMEM_SHARED((N, L), f32),
        ],
    )
    return kern(qg2, kv2, srcp, dstp, eap)


def _edge_body(p_hbm, q_hbm, t_hbm, src_hbm, dst_hbm, out_hbm,
               src_i, dst_i, p_rows, q_rows, t_c, o_c, sem1, sem2):
    c = lax.axis_index("c")
    s = lax.axis_index("s")
    w = s * NC + c

    def _chunk(ch, _):
        base = w * EPW + ch * BC
        pltpu.sync_copy(src_hbm.at[pl.ds(base, BC)], src_i)
        pltpu.sync_copy(dst_hbm.at[pl.ds(base, BC)], dst_i)
        pltpu.sync_copy(t_hbm.at[pl.ds(base, BC)], t_c)
        pltpu.async_copy(p_hbm.at[src_i], p_rows, sem1).wait()
        pltpu.async_copy(q_hbm.at[dst_i], q_rows, sem2).wait()

        def _edge(b, _):
            o_c[b, :] = t_c[b, :] + p_rows[b, :] + q_rows[b, :]
            return 0
        lax.fori_loop(0, BC, _edge, 0)
        pltpu.sync_copy(o_c, out_hbm.at[pl.ds(base, BC)])
        return 0

    lax.fori_loop(0, NCHUNK2, _chunk, 0)


@jax.jit
def _sc_edge(p, q, t, srcp2, dstp2):
    f32 = jnp.float32
    kern = pl.kernel(
        _edge_body,
        out_type=[jax.ShapeDtypeStruct((EP2, DE), f32)],
        mesh=_sc_mesh(),
        scratch_types=[
            pltpu.VMEM((BC,), jnp.int32),
            pltpu.VMEM((BC,), jnp.int32),
            pltpu.VMEM((BC, DE), f32),
            pltpu.VMEM((BC, DE), f32),
            pltpu.VMEM((BC, DE), f32),
            pltpu.VMEM((BC, DE), f32),
            pltpu.SemaphoreType.DMA,
            pltpu.SemaphoreType.DMA,
        ],
    )
    return kern(p, q, t, srcp2, dstp2)[0]


def _erf(x):
    ax = jnp.abs(x)
    t = 1.0 / (1.0 + 0.3275911 * ax)
    poly = t * (0.254829592 + t * (-0.284496736 + t * (1.421413741 +
           t * (-1.453152027 + t * 1.061405429))))
    e = 1.0 - poly * jnp.exp(-ax * ax)
    return jnp.where(x < 0, -e, e)


def _gelu(x):
    return 0.5 * x * (1.0 + _erf(x * np.float32(1.0 / np.sqrt(2.0))))


def _bn(x, g, b, n):
    m = jnp.sum(x, axis=0) / n
    v = jnp.sum(x * x, axis=0) / n - m * m
    return (x - m) * lax.rsqrt(v + 1e-5) * g + b


def kernel(node_attr, edge_attr, Wq, bq, Wk, bk, Wv, bv, We, Wskip, bskip,
           gamma1, beta1, gamma2, beta2, W1, b1, W2, b2, Wep1, bep1, Wep2,
           bep2, gammaE, betaE, edge_index, batch):
    src = edge_index[0]
    dst = edge_index[1]

    blocks_T = [We[:, h * C:(h + 1) * C].T for h in range(H)]
    blocks = [We[:, h * C:(h + 1) * C] for h in range(H)]
    WeTbd = jax.scipy.linalg.block_diag(*blocks_T)
    Webd = jax.scipy.linalg.block_diag(*blocks)
    A = Wep1[DE:DE + D]
    B = Wep1[DE + D:DE + 2 * D]
    Wpq = jnp.concatenate([A + B, B - A], axis=1)
    Wea = Wep1[:DE]

    q = node_attr @ Wq + bq
    k = node_attr @ Wk + bk
    v = node_attr @ Wv + bv
    skipx = node_attr @ Wskip + bskip
    G = q @ WeTbd
    qg2 = jnp.concatenate([
        jnp.concatenate([q[:, :64], G[:, :64]], axis=1),
        jnp.concatenate([q[:, 64:], G[:, 64:]], axis=1)], axis=0)
    kv2 = jnp.concatenate([
        jnp.concatenate([k[:, :64], v[:, :64]], axis=1),
        jnp.concatenate([k[:, 64:], v[:, 64:]], axis=1)], axis=0)

    pad1 = EPT - EPT_RAW
    srcp = jnp.pad(src.reshape(NS, EPT_RAW), ((0, 0), (0, pad1))).reshape(EP)
    dstp = jnp.pad(dst.reshape(NS, EPT_RAW), ((0, 0), (0, pad1))).reshape(EP)
    eap = jnp.pad(edge_attr.reshape(NS, EPT_RAW, DE),
                  ((0, 0), (0, pad1), (0, 0))).reshape(EP, DE)

    outv2, w22, den2 = _sc_attention(qg2, kv2, srcp, dstp, eap)

    outv = jnp.concatenate([outv2[:N], outv2[N:]], axis=1)
    w2f = jnp.concatenate([w22[:N], w22[N:]], axis=1)
    den8 = jnp.concatenate([den2[:N, :4], den2[N:, :4]], axis=1)
    den_rep = jnp.repeat(den8, C, axis=1)

    x = (outv + w2f @ Webd) / (den_rep + 1e-16) + skipx
    y1 = _bn(x, gamma1, beta1, N)
    z = _gelu(y1 @ W1 + b1) @ W2 + b2 + y1
    y2 = _bn(z, gamma2, beta2, N)
    node_out = _gelu(y2)

    PQ = y2 @ Wpq
    P = PQ[:, :DE]
    Qm = PQ[:, DE:]
    T = edge_attr @ Wea + bep1
    pad2 = EPW - EPW_RAW
    srcp2 = jnp.pad(src.reshape(NC * NS, EPW_RAW),
                    ((0, 0), (0, pad2))).reshape(EP2)
    dstp2 = jnp.pad(dst.reshape(NC * NS, EPW_RAW),
                    ((0, 0), (0, pad2))).reshape(EP2)
    tp = jnp.pad(T.reshape(NC * NS, EPW_RAW, DE),
                 ((0, 0), (0, pad2), (0, 0))).reshape(EP2, DE)

    eo_pre = _sc_edge(P, Qm, tp, srcp2, dstp2)
    eo_pre = eo_pre.reshape(NC * NS, EPW, DE)[:, :EPW_RAW].reshape(E, DE)

    u = _gelu(eo_pre) @ Wep2 + bep2
    eo = _gelu(_bn(u, gammaE, betaE, E))
    return (node_out, eo)


# speedup vs baseline: 10.1923x; 10.1923x over previous
import functools

import jax
import jax.numpy as jnp
import numpy as np
from jax import lax
from jax.experimental import pallas as pl
from jax.experimental.pallas import tpu as pltpu
from jax.experimental.pallas import tpu_sc as plsc

N = 10000
E = 320000
D = 128
H = 8
C = 16
DE = 16
DFF = 1024

NC = 2
NS = 16
L = 16

EPT_RAW = E // NS
BCA = 64
BC = 128
NCHUNK = (EPT_RAW + BCA - 1) // BCA
EPT = NCHUNK * BCA
EP = EPT * NS
NP = 10112
ROWS_T = NP // NS

EPW_RAW = E // (NC * NS)
NCHUNK2 = (EPW_RAW + BC - 1) // BC
EPW = NCHUNK2 * BC
EP2 = EPW * NC * NS


def _iota16():
    return lax.broadcasted_iota(jnp.int32, (L,), 0)


def _sc_mesh():
    return plsc.VectorSubcoreMesh(
        core_axis_name="c", subcore_axis_name="s", num_cores=NC,
        num_subcores=NS)


def _attn_body(qg_hbm, kv_hbm, src_hbm, dst_hbm, ea_hbm,
               outv_hbm, w2_hbm, den_hbm,
               src_i, dst_i, src_g, dst_g, kv_rows, qg_rows, ea_c,
               msg_c, w2_c, den_c, sem1, sem2,
               sh_outv, sh_w2, sh_den):
    c = lax.axis_index("c")
    s = lax.axis_index("s")
    zero16 = jnp.zeros((L,), jnp.float32)

    def _z64(i, _):
        for jj in range(4):
            msg_c[i, pl.ds(jj * L, L)] = zero16
        den_c[i, :] = zero16
        return 0
    lax.fori_loop(0, BCA, _z64, 0)

    for t in range(10):
        r0 = s * ROWS_T + t * BCA
        nr = BCA if t < 9 else ROWS_T - 9 * BCA
        pltpu.sync_copy(msg_c.at[pl.ds(0, nr)], sh_outv.at[pl.ds(r0, nr)])
        pltpu.sync_copy(msg_c.at[pl.ds(0, nr)], sh_w2.at[pl.ds(r0, nr)])
        pltpu.sync_copy(den_c.at[pl.ds(0, nr)], sh_den.at[pl.ds(r0, nr)])
    plsc.subcore_barrier()

    coff = c * N
    iota = _iota16()

    def _chunk(ch, _):
        base = s * EPT + ch * BCA
        pltpu.sync_copy(src_hbm.at[pl.ds(base, BCA)], src_i)
        pltpu.sync_copy(dst_hbm.at[pl.ds(base, BCA)], dst_i)
        pltpu.sync_copy(ea_hbm.at[pl.ds(base, BCA)], ea_c)
        for g in range(BCA // L):
            sl = pl.ds(g * L, L)
            src_g[sl] = src_i[sl] + coff
            dst_g[sl] = dst_i[sl] + coff
        pltpu.async_copy(kv_hbm.at[src_g], kv_rows, sem1).wait()
        pltpu.async_copy(qg_hbm.at[dst_g], qg_rows, sem2).wait()

        def _edge(b, _):
            ea_b = ea_c[b, :]
            validv = jnp.broadcast_to(ch * BCA + b, (L,)) < EPT_RAW
            exs = []
            for h in range(4):
                qd = qg_rows[b, pl.ds(h * L, L)]
                ks = kv_rows[b, pl.ds(h * L, L)]
                gh = qg_rows[b, pl.ds(64 + h * L, L)]
                a_h = jnp.sum(qd * ks + gh * ea_b)
                av = jnp.broadcast_to(a_h, (L,))
                exb = jnp.where(validv,
                                jnp.exp(jnp.minimum(av * 0.25, 60.0)), 0.0)
                exs.append(exb)
                v_b = kv_rows[b, pl.ds(64 + h * L, L)]
                msg_c[b, pl.ds(h * L, L)] = exb * v_b
                w2_c[b, pl.ds(h * L, L)] = exb * ea_b
            denv = zero16
            for h in range(4):
                denv = jnp.where(iota == h, exs[h], denv)
            den_c[b, :] = denv
            return 0
        lax.fori_loop(0, BCA, _edge, 0)

        pltpu.sync_copy(msg_c, sh_outv.at[dst_i], add=True)
        pltpu.sync_copy(w2_c, sh_w2.at[dst_i], add=True)
        pltpu.sync_copy(den_c, sh_den.at[dst_i], add=True)
        return 0

    lax.fori_loop(0, NCHUNK, _chunk, 0)
    plsc.subcore_barrier()

    r0 = s * ROWS_T
    o0 = c * NP + s * ROWS_T
    pltpu.sync_copy(sh_outv.at[pl.ds(r0, ROWS_T)], outv_hbm.at[pl.ds(o0, ROWS_T)])
    pltpu.sync_copy(sh_w2.at[pl.ds(r0, ROWS_T)], w2_hbm.at[pl.ds(o0, ROWS_T)])
    pltpu.sync_copy(sh_den.at[pl.ds(r0, ROWS_T)], den_hbm.at[pl.ds(o0, ROWS_T)])


@jax.jit
def _sc_attention(qg2, kv2, srcp, dstp, eap):
    f32 = jnp.float32
    kern = pl.kernel(
        _attn_body,
        out_type=[
            jax.ShapeDtypeStruct((NC * NP, 64), f32),
            jax.ShapeDtypeStruct((NC * NP, 64), f32),
            jax.ShapeDtypeStruct((NC * NP, L), f32),
        ],
        mesh=_sc_mesh(),
        compiler_params=pltpu.CompilerParams(needs_layout_passes=False, use_tc_tiling_on_sc=False),
        scratch_types=[
            pltpu.VMEM((BCA,), jnp.int32),
            pltpu.VMEM((BCA,), jnp.int32),
            pltpu.VMEM((BCA,), jnp.int32),
            pltpu.VMEM((BCA,), jnp.int32),
            pltpu.VMEM((BCA, 128), f32),
            pltpu.VMEM((BCA, 128), f32),
            pltpu.VMEM((BCA, DE), f32),
            pltpu.VMEM((BCA, 64), f32),
            pltpu.VMEM((BCA, 64), f32),
            pltpu.VMEM((BCA, L), f32),
            pltpu.SemaphoreType.DMA,
            pltpu.SemaphoreType.DMA,
            pltpu.VMEM_SHARED((NP, 64), f32),
            pltpu.VMEM_SHARED((NP, 64), f32),
            pltpu.VMEM_SHARED((NP, L), f32),
        ],
    )
    return kern(qg2, kv2, srcp, dstp, eap)


def _edge_body(p_hbm, q_hbm, t_hbm, src_hbm, dst_hbm, out_hbm,
               src_i, dst_i, p_rows, q_rows, t_c, o_c, sem1, sem2):
    c = lax.axis_index("c")
    s = lax.axis_index("s")
    w = s * NC + c

    def _chunk(ch, _):
        base = w * EPW + ch * BC
        pltpu.sync_copy(src_hbm.at[pl.ds(base, BC)], src_i)
        pltpu.sync_copy(dst_hbm.at[pl.ds(base, BC)], dst_i)
        pltpu.sync_copy(t_hbm.at[pl.ds(base, BC)], t_c)
        pltpu.async_copy(p_hbm.at[src_i], p_rows, sem1).wait()
        pltpu.async_copy(q_hbm.at[dst_i], q_rows, sem2).wait()

        def _edge(b, _):
            o_c[b, :] = t_c[b, :] + p_rows[b, :] + q_rows[b, :]
            return 0
        lax.fori_loop(0, BC, _edge, 0)
        pltpu.sync_copy(o_c, out_hbm.at[pl.ds(base, BC)])
        return 0

    lax.fori_loop(0, NCHUNK2, _chunk, 0)


@jax.jit
def _sc_edge(p, q, t, srcp2, dstp2):
    f32 = jnp.float32
    kern = pl.kernel(
        _edge_body,
        out_type=[jax.ShapeDtypeStruct((EP2, DE), f32)],
        mesh=_sc_mesh(),
        compiler_params=pltpu.CompilerParams(needs_layout_passes=False, use_tc_tiling_on_sc=False),
        scratch_types=[
            pltpu.VMEM((BC,), jnp.int32),
            pltpu.VMEM((BC,), jnp.int32),
            pltpu.VMEM((BC, DE), f32),
            pltpu.VMEM((BC, DE), f32),
            pltpu.VMEM((BC, DE), f32),
            pltpu.VMEM((BC, DE), f32),
            pltpu.SemaphoreType.DMA,
            pltpu.SemaphoreType.DMA,
        ],
    )
    return kern(p, q, t, srcp2, dstp2)[0]


def _erf(x):
    ax = jnp.abs(x)
    t = 1.0 / (1.0 + 0.3275911 * ax)
    poly = t * (0.254829592 + t * (-0.284496736 + t * (1.421413741 +
           t * (-1.453152027 + t * 1.061405429))))
    e = 1.0 - poly * jnp.exp(-ax * ax)
    return jnp.where(x < 0, -e, e)


def _gelu(x):
    return 0.5 * x * (1.0 + _erf(x * np.float32(1.0 / np.sqrt(2.0))))


def _bn(x, g, b, n):
    m = jnp.sum(x, axis=0) / n
    v = jnp.sum(x * x, axis=0) / n - m * m
    return (x - m) * lax.rsqrt(v + 1e-5) * g + b


def kernel(node_attr, edge_attr, Wq, bq, Wk, bk, Wv, bv, We, Wskip, bskip,
           gamma1, beta1, gamma2, beta2, W1, b1, W2, b2, Wep1, bep1, Wep2,
           bep2, gammaE, betaE, edge_index, batch):
    src = edge_index[0]
    dst = edge_index[1]

    blocks_T = [We[:, h * C:(h + 1) * C].T for h in range(H)]
    blocks = [We[:, h * C:(h + 1) * C] for h in range(H)]
    WeTbd = jax.scipy.linalg.block_diag(*blocks_T)
    Webd = jax.scipy.linalg.block_diag(*blocks)
    A = Wep1[DE:DE + D]
    B = Wep1[DE + D:DE + 2 * D]
    Wpq = jnp.concatenate([A + B, B - A], axis=1)
    Wea = Wep1[:DE]

    q = node_attr @ Wq + bq
    k = node_attr @ Wk + bk
    v = node_attr @ Wv + bv
    skipx = node_attr @ Wskip + bskip
    G = q @ WeTbd
    qg2 = jnp.concatenate([
        jnp.concatenate([q[:, :64], G[:, :64]], axis=1),
        jnp.concatenate([q[:, 64:], G[:, 64:]], axis=1)], axis=0)
    kv2 = jnp.concatenate([
        jnp.concatenate([k[:, :64], v[:, :64]], axis=1),
        jnp.concatenate([k[:, 64:], v[:, 64:]], axis=1)], axis=0)

    pad1 = EPT - EPT_RAW
    srcp = jnp.pad(src.reshape(NS, EPT_RAW), ((0, 0), (0, pad1))).reshape(EP)
    dstp = jnp.pad(dst.reshape(NS, EPT_RAW), ((0, 0), (0, pad1))).reshape(EP)
    eap = jnp.pad(edge_attr.reshape(NS, EPT_RAW, DE),
                  ((0, 0), (0, pad1), (0, 0))).reshape(EP, DE)

    outv2, w22, den2 = _sc_attention(qg2, kv2, srcp, dstp, eap)

    outv = jnp.concatenate([outv2[:N], outv2[NP:NP + N]], axis=1)
    w2f = jnp.concatenate([w22[:N], w22[NP:NP + N]], axis=1)
    den8 = jnp.concatenate([den2[:N, :4], den2[NP:NP + N, :4]], axis=1)
    den_rep = jnp.repeat(den8, C, axis=1)

    x = (outv + w2f @ Webd) / (den_rep + 1e-16) + skipx
    y1 = _bn(x, gamma1, beta1, N)
    z = _gelu(y1 @ W1 + b1) @ W2 + b2 + y1
    y2 = _bn(z, gamma2, beta2, N)
    node_out = _gelu(y2)

    PQ = y2 @ Wpq
    P = PQ[:, :DE]
    Qm = PQ[:, DE:]
    T = edge_attr @ Wea + bep1
    pad2 = EPW - EPW_RAW
    srcp2 = jnp.pad(src.reshape(NC * NS, EPW_RAW),
                    ((0, 0), (0, pad2))).reshape(EP2)
    dstp2 = jnp.pad(dst.reshape(NC * NS, EPW_RAW),
                    ((0, 0), (0, pad2))).reshape(EP2)
    tp = jnp.pad(T.reshape(NC * NS, EPW_RAW, DE),
                 ((0, 0), (0, pad2), (0, 0))).reshape(EP2, DE)

    eo_pre = _sc_edge(P, Qm, tp, srcp2, dstp2)
    eo_pre = eo_pre.reshape(NC * NS, EPW, DE)[:, :EPW_RAW].reshape(E, DE)

    u = _gelu(eo_pre) @ Wep2 + bep2
    eo = _gelu(_bn(u, gammaE, betaE, E))
    return (node_out, eo)


# speedup vs baseline: 17.2371x; 1.6912x over previous
import functools

import jax
import jax.numpy as jnp
import numpy as np
from jax import lax
from jax.experimental import pallas as pl
from jax.experimental.pallas import tpu as pltpu
from jax.experimental.pallas import tpu_sc as plsc

N = 10000
E = 320000
D = 128
H = 8
C = 16
DE = 16
DFF = 1024

NC = 2
NS = 16
L = 16
PAD = 128

EPT_RAW = E // NS
BCA = 128
NCHUNK = (EPT_RAW + BCA - 1) // BCA
NP = 10112
ROWS_T = NP // NS
AW = 80

EPW_RAW = E // (NC * NS)
BC = 128
NCHUNK2 = (EPW_RAW + BC - 1) // BC

BN_N = 1000
BN_E = 2560


def _iota16():
    return lax.broadcasted_iota(jnp.int32, (L,), 0)


def _sc_mesh():
    return plsc.VectorSubcoreMesh(
        core_axis_name="c", subcore_axis_name="s", num_cores=NC,
        num_subcores=NS)


_SC_PARAMS = dict(
    compiler_params=pltpu.CompilerParams(
        needs_layout_passes=False, use_tc_tiling_on_sc=False))


def _attn_body(q_hbm, kv_hbm, e_hbm, src_hbm, dst_hbm,
               acc_hbm,
               src_i0, dst_i0, src_g0, dst_g0, q_rows0, kv_rows0, e_c0,
               msg_c0,
               src_i1, dst_i1, src_g1, dst_g1, q_rows1, kv_rows1,
               msg_c1,
               sem_kv0, sem_q0, sem_sc0, sem_kv1, sem_q1, sem_sc1,
               sh_acc):
    c = lax.axis_index("c")
    s = lax.axis_index("s")
    zero16 = jnp.zeros((L,), jnp.float32)
    iota = _iota16()

    def _z(i, _):
        for jj in range(AW // L):
            msg_c0[i, pl.ds(jj * L, L)] = zero16
        return 0
    lax.fori_loop(0, BCA, _z, 0)

    for t in range(5):
        r0 = s * ROWS_T + t * BCA
        nr = BCA if t < 4 else ROWS_T - 4 * BCA
        pltpu.sync_copy(msg_c0.at[pl.ds(0, nr)], sh_acc.at[pl.ds(r0, nr)])
    plsc.subcore_barrier()

    coff = c * N
    eoff = c * E

    slots = (
        (src_i0, dst_i0, src_g0, dst_g0, q_rows0, kv_rows0, e_c0, msg_c0,
         sem_kv0, sem_q0, sem_sc0),
        (src_i1, dst_i1, src_g1, dst_g1, q_rows1, kv_rows1, e_c0, msg_c1,
         sem_kv1, sem_q1, sem_sc1),
    )

    def _load(ch, slot):
        (src_i, dst_i, src_g, dst_g, q_rows, kv_rows, e_c, msg_c,
         sem_kv, sem_q, sem_sc) = slots[slot]
        base = s * EPT_RAW + ch * BCA
        pltpu.sync_copy(src_hbm.at[pl.ds(base, BCA)], src_i)
        pltpu.sync_copy(dst_hbm.at[pl.ds(base, BCA)], dst_i)
        for g in range(BCA // L):
            sl = pl.ds(g * L, L)
            src_g[sl] = src_i[sl] + coff
            dst_g[sl] = dst_i[sl] + coff
        pltpu.async_copy(kv_hbm.at[src_g], kv_rows, sem_kv)
        pltpu.async_copy(q_hbm.at[dst_g], q_rows, sem_q)

    def _wait_sc(slot):
        (src_i, dst_i, src_g, dst_g, q_rows, kv_rows, e_c, msg_c,
         sem_kv, sem_q, sem_sc) = slots[slot]
        pltpu.make_async_copy(msg_c, sh_acc.at[dst_i], sem_sc).wait()

    def _compute(ch, slot):
        (src_i, dst_i, src_g, dst_g, q_rows, kv_rows, e_c, msg_c,
         sem_kv, sem_q, sem_sc) = slots[slot]
        base = s * EPT_RAW + ch * BCA
        pltpu.sync_copy(e_hbm.at[pl.ds(eoff + base, BCA)], e_c)
        pltpu.make_async_copy(kv_hbm.at[src_g], kv_rows, sem_kv).wait()
        pltpu.make_async_copy(q_hbm.at[dst_g], q_rows, sem_q).wait()

        @plsc.parallel_loop(0, BCA, unroll=2)
        def _edge(b):
            validv = jnp.broadcast_to(ch * BCA + b, (L,)) < EPT_RAW
            exs = []
            for h in range(4):
                qd = q_rows[b, pl.ds(h * L, L)]
                kh = kv_rows[b, pl.ds(h * L, L)]
                eh = e_c[b, pl.ds(h * L, L)]
                a_h = jnp.sum(qd * (kh + eh))
                av = jnp.broadcast_to(a_h, (L,))
                exb = jnp.where(validv,
                                jnp.exp(jnp.minimum(av * 0.25, 60.0)), 0.0)
                vh = kv_rows[b, pl.ds(64 + h * L, L)]
                msg_c[b, pl.ds(h * L, L)] = exb * (vh + eh)
                exs.append(exb)
            denv = zero16
            for h in range(4):
                denv = jnp.where(iota == h, exs[h], denv)
            msg_c[b, pl.ds(64, L)] = denv

        pltpu.async_copy(msg_c, sh_acc.at[dst_i], sem_sc, add=True)

    _load(0, 0)
    NPAIR = (NCHUNK + 1) // 2

    def _pair(i, _):
        ch0 = 2 * i
        ch1 = ch0 + 1

        @pl.when(jnp.logical_and(ch1 < NCHUNK, i > 0))
        def _():
            _wait_sc(1)

        @pl.when(ch1 < NCHUNK)
        def _():
            _load(ch1, 1)
        _compute(ch0, 0)

        @pl.when(ch0 + 2 < NCHUNK)
        def _():
            _wait_sc(0)
            _load(ch0 + 2, 0)

        @pl.when(ch1 < NCHUNK)
        def _():
            _compute(ch1, 1)
        return 0

    lax.fori_loop(0, NPAIR, _pair, 0)
    _wait_sc(0)
    _wait_sc(1)
    plsc.subcore_barrier()

    r0 = s * ROWS_T
    o0 = c * NP + s * ROWS_T
    pltpu.sync_copy(sh_acc.at[pl.ds(r0, ROWS_T)], acc_hbm.at[pl.ds(o0, ROWS_T)])


@jax.jit
def _sc_attention(q2, kv2, e2, srcp, dstp):
    f32 = jnp.float32
    kern = pl.kernel(
        _attn_body,
        out_type=[jax.ShapeDtypeStruct((NC * NP, AW), f32)],
        mesh=_sc_mesh(),
        scratch_types=(
            [pltpu.VMEM((BCA,), jnp.int32)] * 4
            + [pltpu.VMEM((BCA, 64), f32), pltpu.VMEM((BCA, 128), f32),
               pltpu.VMEM((BCA, 64), f32), pltpu.VMEM((BCA, AW), f32)]
            + [pltpu.VMEM((BCA,), jnp.int32)] * 4
            + [pltpu.VMEM((BCA, 64), f32), pltpu.VMEM((BCA, 128), f32),
               pltpu.VMEM((BCA, AW), f32)]
            + [pltpu.SemaphoreType.DMA] * 6
            + [pltpu.VMEM_SHARED((NP, AW), f32)]
        ),
        **_SC_PARAMS,
    )
    return kern(q2, kv2, e2, srcp, dstp)[0]


def _edge_body(p_hbm, q_hbm, t_hbm, src_hbm, dst_hbm, out_hbm,
               src_i, dst_i, p_rows, q_rows, t_c, o_c, sem1, sem2):
    c = lax.axis_index("c")
    s = lax.axis_index("s")
    w = s * NC + c

    def _chunk(ch, _):
        base = w * EPW_RAW + ch * BC
        pltpu.sync_copy(src_hbm.at[pl.ds(base, BC)], src_i)
        pltpu.sync_copy(dst_hbm.at[pl.ds(base, BC)], dst_i)
        pltpu.sync_copy(t_hbm.at[pl.ds(base, BC)], t_c)
        pltpu.async_copy(p_hbm.at[src_i], p_rows, sem1).wait()
        pltpu.async_copy(q_hbm.at[dst_i], q_rows, sem2).wait()

        def _edge(b, _):
            o_c[b, :] = t_c[b, :] + p_rows[b, :] + q_rows[b, :]
            return 0
        lax.fori_loop(0, BC, _edge, 0)
        pltpu.sync_copy(o_c, out_hbm.at[pl.ds(base, BC)])
        return 0

    lax.fori_loop(0, NCHUNK2, _chunk, 0)


@jax.jit
def _sc_edge(p, q, t, srcp2, dstp2):
    f32 = jnp.float32
    kern = pl.kernel(
        _edge_body,
        out_type=[jax.ShapeDtypeStruct((E + PAD, DE), f32)],
        mesh=_sc_mesh(),
        scratch_types=[
            pltpu.VMEM((BC,), jnp.int32),
            pltpu.VMEM((BC,), jnp.int32),
            pltpu.VMEM((BC, DE), f32),
            pltpu.VMEM((BC, DE), f32),
            pltpu.VMEM((BC, DE), f32),
            pltpu.VMEM((BC, DE), f32),
            pltpu.SemaphoreType.DMA,
            pltpu.SemaphoreType.DMA,
        ],
        **_SC_PARAMS,
    )
    return kern(p, q, t, srcp2, dstp2)[0]


def _erf(x):
    ax = jnp.abs(x)
    t = 1.0 / (1.0 + 0.3275911 * ax)
    poly = t * (0.254829592 + t * (-0.284496736 + t * (1.421413741 +
           t * (-1.453152027 + t * 1.061405429))))
    e = 1.0 - poly * jnp.exp(-ax * ax)
    return jnp.where(x < 0, -e, e)


def _gelu(x):
    return 0.5 * x * (1.0 + _erf(x * np.float32(1.0 / np.sqrt(2.0))))


def _stats_rows(xb):
    s0 = jnp.sum(xb, axis=0, keepdims=True)
    s1 = jnp.sum(xb * xb, axis=0, keepdims=True)
    return jnp.concatenate([s0, s1, jnp.zeros((6, xb.shape[1]), jnp.float32)], 0)


def _kpre_body(na, wall, ball, q_o, kv_o, skip_o):
    x = na[...]
    qkvs = x @ wall[...] + ball[...]
    q = qkvs[:, :128]
    k = qkvs[:, 128:256]
    v = qkvs[:, 256:384]
    skip_o[...] = qkvs[:, 384:512]
    q_o[0] = q[:, :64]
    q_o[1] = q[:, 64:]
    kv_o[0] = jnp.concatenate([k[:, :64], v[:, :64]], axis=1)
    kv_o[1] = jnp.concatenate([k[:, 64:], v[:, 64:]], axis=1)


def k_pre(na, wall, ball):
    f32 = jnp.float32
    grid = N // BN_N
    return pl.pallas_call(
        _kpre_body,
        grid=(grid,),
        in_specs=[
            pl.BlockSpec((BN_N, 128), lambda i: (i, 0)),
            pl.BlockSpec((128, 512), lambda i: (0, 0)),
            pl.BlockSpec((1, 512), lambda i: (0, 0)),
        ],
        out_specs=[
            pl.BlockSpec((2, BN_N, 64), lambda i: (0, i, 0)),
            pl.BlockSpec((2, BN_N, 128), lambda i: (0, i, 0)),
            pl.BlockSpec((BN_N, 128), lambda i: (i, 0)),
        ],
        out_shape=[
            jax.ShapeDtypeStruct((2, N, 64), f32),
            jax.ShapeDtypeStruct((2, N, 128), f32),
            jax.ShapeDtypeStruct((N, 128), f32),
        ],
    )(na, wall, ball)


def _ke_body(ea, we, e_o):
    e = ea[...] @ we[...]
    e_o[0] = e[:, :64]
    e_o[1] = e[:, 64:]


def k_e(ea, we):
    f32 = jnp.float32
    grid = E // BN_E
    return pl.pallas_call(
        _ke_body,
        grid=(grid,),
        in_specs=[
            pl.BlockSpec((BN_E, 16), lambda i: (i, 0)),
            pl.BlockSpec((16, 128), lambda i: (0, 0)),
        ],
        out_specs=[pl.BlockSpec((2, BN_E, 64), lambda i: (0, i, 0))],
        out_shape=[jax.ShapeDtypeStruct((2, E, 64), f32)],
    )(ea, we)[0]


def _k1_body(outv, denr, skipx, x_o, st_o):
    i = pl.program_id(0)
    x = outv[...] / (denr[...] + 1e-16) + skipx[...]
    x_o[...] = x
    rows = _stats_rows(x)

    @pl.when(i == 0)
    def _():
        st_o[...] = rows

    @pl.when(i > 0)
    def _():
        st_o[...] = st_o[...] + rows


def k1(outv, denr, skipx):
    f32 = jnp.float32
    grid = N // BN_N
    return pl.pallas_call(
        _k1_body,
        grid=(grid,),
        in_specs=[
            pl.BlockSpec((BN_N, 128), lambda i: (i, 0)),
            pl.BlockSpec((BN_N, 128), lambda i: (i, 0)),
            pl.BlockSpec((BN_N, 128), lambda i: (i, 0)),
        ],
        out_specs=[
            pl.BlockSpec((BN_N, 128), lambda i: (i, 0)),
            pl.BlockSpec((8, 128), lambda i: (0, 0)),
        ],
        out_shape=[
            jax.ShapeDtypeStruct((N, 128), f32),
            jax.ShapeDtypeStruct((8, 128), f32),
        ],
    )(outv, denr, skipx)


def _k2_body(x, st1, g1, b1n, w1, b1f, w2, b2f, z_o, st_o):
    i = pl.program_id(0)
    mean = st1[0:1, :] * np.float32(1.0 / N)
    var = st1[1:2, :] * np.float32(1.0 / N) - mean * mean
    y1 = (x[...] - mean) * lax.rsqrt(var + 1e-5) * g1[...] + b1n[...]
    h = _gelu(y1 @ w1[...] + b1f[...])
    z = h @ w2[...] + b2f[...] + y1
    z_o[...] = z
    rows = _stats_rows(z)

    @pl.when(i == 0)
    def _():
        st_o[...] = rows

    @pl.when(i > 0)
    def _():
        st_o[...] = st_o[...] + rows


def k2(x, st1, g1, b1n, w1, b1f, w2, b2f):
    f32 = jnp.float32
    grid = N // BN_N
    return pl.pallas_call(
        _k2_body,
        grid=(grid,),
        in_specs=[
            pl.BlockSpec((BN_N, 128), lambda i: (i, 0)),
            pl.BlockSpec((8, 128), lambda i: (0, 0)),
            pl.BlockSpec((1, 128), lambda i: (0, 0)),
            pl.BlockSpec((1, 128), lambda i: (0, 0)),
            pl.BlockSpec((128, 1024), lambda i: (0, 0)),
            pl.BlockSpec((1, 1024), lambda i: (0, 0)),
            pl.BlockSpec((1024, 128), lambda i: (0, 0)),
            pl.BlockSpec((1, 128), lambda i: (0, 0)),
        ],
        out_specs=[
            pl.BlockSpec((BN_N, 128), lambda i: (i, 0)),
            pl.BlockSpec((8, 128), lambda i: (0, 0)),
        ],
        out_shape=[
            jax.ShapeDtypeStruct((N, 128), f32),
            jax.ShapeDtypeStruct((8, 128), f32),
        ],
    )(x, st1, g1, b1n, w1, b1f, w2, b2f)


def _k3_body(z, st2, g2, b2n, wpq, no_o, pq_o):
    mean = st2[0:1, :] * np.float32(1.0 / N)
    var = st2[1:2, :] * np.float32(1.0 / N) - mean * mean
    y2 = (z[...] - mean) * lax.rsqrt(var + 1e-5) * g2[...] + b2n[...]
    no_o[...] = _gelu(y2)
    pq_o[...] = y2 @ wpq[...]


def k3(z, st2, g2, b2n, wpq):
    f32 = jnp.float32
    grid = N // BN_N
    return pl.pallas_call(
        _k3_body,
        grid=(grid,),
        in_specs=[
            pl.BlockSpec((BN_N, 128), lambda i: (i, 0)),
            pl.BlockSpec((8, 128), lambda i: (0, 0)),
            pl.BlockSpec((1, 128), lambda i: (0, 0)),
            pl.BlockSpec((1, 128), lambda i: (0, 0)),
            pl.BlockSpec((128, 32), lambda i: (0, 0)),
        ],
        out_specs=[
            pl.BlockSpec((BN_N, 128), lambda i: (i, 0)),
            pl.BlockSpec((BN_N, 32), lambda i: (i, 0)),
        ],
        out_shape=[
            jax.ShapeDtypeStruct((N, 128), f32),
            jax.ShapeDtypeStruct((N, 32), f32),
        ],
    )(z, st2, g2, b2n, wpq)


def _kt_body(ea, wea, bep1, t_o):
    t_o[...] = ea[...] @ wea[...] + bep1[...]


def k_t(ea, wea, bep1):
    f32 = jnp.float32
    grid = E // BN_E
    return pl.pallas_call(
        _kt_body,
        grid=(grid,),
        in_specs=[
            pl.BlockSpec((BN_E, 16), lambda i: (i, 0)),
            pl.BlockSpec((16, 16), lambda i: (0, 0)),
            pl.BlockSpec((1, 16), lambda i: (0, 0)),
        ],
        out_specs=[pl.BlockSpec((BN_E, 16), lambda i: (i, 0))],
        out_shape=[jax.ShapeDtypeStruct((E, 16), f32)],
    )(ea, wea, bep1)[0]


def _k5a_body(eop, wep2, bep2, u_o, st_o):
    i = pl.program_id(0)
    u = _gelu(eop[...]) @ wep2[...] + bep2[...]
    u_o[...] = u
    rows = _stats_rows(u)

    @pl.when(i == 0)
    def _():
        st_o[...] = rows

    @pl.when(i > 0)
    def _():
        st_o[...] = st_o[...] + rows


def k5a(eop, wep2, bep2):
    f32 = jnp.float32
    grid = E // BN_E
    return pl.pallas_call(
        _k5a_body,
        grid=(grid,),
        in_specs=[
            pl.BlockSpec((BN_E, 16), lambda i: (i, 0)),
            pl.BlockSpec((16, 16), lambda i: (0, 0)),
            pl.BlockSpec((1, 16), lambda i: (0, 0)),
        ],
        out_specs=[
            pl.BlockSpec((BN_E, 16), lambda i: (i, 0)),
            pl.BlockSpec((8, 16), lambda i: (0, 0)),
        ],
        out_shape=[
            jax.ShapeDtypeStruct((E, 16), f32),
            jax.ShapeDtypeStruct((8, 16), f32),
        ],
    )(eop, wep2, bep2)


def _k5b_body(u, ste, ge, be, eo_o):
    mean = ste[0:1, :] * np.float32(1.0 / E)
    var = ste[1:2, :] * np.float32(1.0 / E) - mean * mean
    y = (u[...] - mean) * lax.rsqrt(var + 1e-5) * ge[...] + be[...]
    eo_o[...] = _gelu(y)


def k5b(u, ste, ge, be):
    f32 = jnp.float32
    grid = E // BN_E
    return pl.pallas_call(
        _k5b_body,
        grid=(grid,),
        in_specs=[
            pl.BlockSpec((BN_E, 16), lambda i: (i, 0)),
            pl.BlockSpec((8, 16), lambda i: (0, 0)),
            pl.BlockSpec((1, 16), lambda i: (0, 0)),
            pl.BlockSpec((1, 16), lambda i: (0, 0)),
        ],
        out_specs=[pl.BlockSpec((BN_E, 16), lambda i: (i, 0))],
        out_shape=[jax.ShapeDtypeStruct((E, 16), f32)],
    )(u, ste, ge, be)[0]


def kernel(node_attr, edge_attr, Wq, bq, Wk, bk, Wv, bv, We, Wskip, bskip,
           gamma1, beta1, gamma2, beta2, W1, b1, W2, b2, Wep1, bep1, Wep2,
           bep2, gammaE, betaE, edge_index, batch):
    src = edge_index[0]
    dst = edge_index[1]

    A = Wep1[DE:DE + D]
    B = Wep1[DE + D:DE + 2 * D]
    Wpq = jnp.concatenate([A + B, B - A], axis=1)
    Wea = Wep1[:DE]

    Wall = jnp.concatenate([Wq, Wk, Wv, Wskip], axis=1)
    ball = jnp.concatenate([bq, bk, bv, bskip]).reshape(1, 512)
    q3, kv3, skipx = k_pre(node_attr, Wall, ball)
    q2 = q3.reshape(NC * N, 64)
    kv2 = kv3.reshape(NC * N, 128)
    e2 = jnp.pad(k_e(edge_attr, We).reshape(NC * E, 64), ((0, PAD), (0, 0)))

    srcp = jnp.pad(src, (0, PAD))
    dstp = jnp.pad(dst, (0, PAD))

    acc = _sc_attention(q2, kv2, e2, srcp, dstp)
    outv = jnp.concatenate([acc[:N, :64], acc[NP:NP + N, :64]], axis=1)
    den8 = jnp.concatenate([acc[:N, 64:68], acc[NP:NP + N, 64:68]], axis=1)
    den_rep = jnp.repeat(den8, C, axis=1)

    x, st1 = k1(outv, den_rep, skipx)
    z, st2 = k2(x, st1, gamma1.reshape(1, 128), beta1.reshape(1, 128),
                W1, b1.reshape(1, DFF), W2, b2.reshape(1, 128))
    node_out, PQ = k3(z, st2, gamma2.reshape(1, 128), beta2.reshape(1, 128),
                      Wpq)

    P = PQ[:, :DE]
    Qm = PQ[:, DE:]
    T = jnp.pad(k_t(edge_attr, Wea, bep1.reshape(1, DE)), ((0, PAD), (0, 0)))

    eo_pre = _sc_edge(P, Qm, T, srcp, dstp)[:E]
    u, stE = k5a(eo_pre, Wep2, bep2.reshape(1, DE))
    eo = k5b(u, stE, gammaE.reshape(1, DE), betaE.reshape(1, DE))
    return (node_out, eo)


# speedup vs baseline: 20.2219x; 1.1732x over previous
import functools

import jax
import jax.numpy as jnp
import numpy as np
from jax import lax
from jax.experimental import pallas as pl
from jax.experimental.pallas import tpu as pltpu
from jax.experimental.pallas import tpu_sc as plsc

N = 10000
E = 320000
D = 128
H = 8
C = 16
DE = 16
DFF = 1024

NC = 2
NS = 16
L = 16
PAD = 128

EPT_RAW = E // NS
BCA = 128
NCHUNK = (EPT_RAW + BCA - 1) // BCA
NP = 10112
ROWS_T = NP // NS
AW = 80

EPW_RAW = E // (NC * NS)
BC = 128
NCHUNK2 = (EPW_RAW + BC - 1) // BC
ES = E + 2560

BN_N = 1000
BN_E = 2560
EP8 = E // 8
BN_P = 4000


def _iota16():
    return lax.broadcasted_iota(jnp.int32, (L,), 0)


def _sc_mesh():
    return plsc.VectorSubcoreMesh(
        core_axis_name="c", subcore_axis_name="s", num_cores=NC,
        num_subcores=NS)


_SC_PARAMS = dict(
    compiler_params=pltpu.CompilerParams(
        needs_layout_passes=False, use_tc_tiling_on_sc=False))


def _attn_body(q_hbm, kv_hbm, e_hbm, src_hbm, dst_hbm,
               acc_hbm,
               src_i0, dst_i0, src_g0, dst_g0, q_rows0, kv_rows0, e_c0,
               msg_c0,
               src_i1, dst_i1, src_g1, dst_g1, q_rows1, kv_rows1,
               msg_c1,
               sem_kv0, sem_q0, sem_sc0, sem_kv1, sem_q1, sem_sc1,
               sh_acc):
    c = lax.axis_index("c")
    s = lax.axis_index("s")
    zero16 = jnp.zeros((L,), jnp.float32)
    iota = _iota16()

    def _z(i, _):
        for jj in range(AW // L):
            msg_c0[i, pl.ds(jj * L, L)] = zero16
        return 0
    lax.fori_loop(0, BCA, _z, 0)

    for t in range(5):
        r0 = s * ROWS_T + t * BCA
        nr = BCA if t < 4 else ROWS_T - 4 * BCA
        pltpu.sync_copy(msg_c0.at[pl.ds(0, nr)], sh_acc.at[pl.ds(r0, nr)])
    plsc.subcore_barrier()

    coff = c * N
    eoff = c * ES

    slots = (
        (src_i0, dst_i0, src_g0, dst_g0, q_rows0, kv_rows0, e_c0, msg_c0,
         sem_kv0, sem_q0, sem_sc0),
        (src_i1, dst_i1, src_g1, dst_g1, q_rows1, kv_rows1, e_c0, msg_c1,
         sem_kv1, sem_q1, sem_sc1),
    )

    def _load(ch, slot):
        (src_i, dst_i, src_g, dst_g, q_rows, kv_rows, e_c, msg_c,
         sem_kv, sem_q, sem_sc) = slots[slot]
        base = s * EPT_RAW + ch * BCA
        pltpu.sync_copy(src_hbm.at[pl.ds(base, BCA)], src_i)
        pltpu.sync_copy(dst_hbm.at[pl.ds(base, BCA)], dst_i)
        for g in range(BCA // L):
            sl = pl.ds(g * L, L)
            src_g[sl] = src_i[sl] + coff
            dst_g[sl] = dst_i[sl] + coff
        pltpu.async_copy(kv_hbm.at[src_g], kv_rows, sem_kv)
        pltpu.async_copy(q_hbm.at[dst_g], q_rows, sem_q)

    def _wait_sc(slot):
        (src_i, dst_i, src_g, dst_g, q_rows, kv_rows, e_c, msg_c,
         sem_kv, sem_q, sem_sc) = slots[slot]
        pltpu.make_async_copy(msg_c, sh_acc.at[dst_i], sem_sc).wait()

    def _compute(ch, slot):
        (src_i, dst_i, src_g, dst_g, q_rows, kv_rows, e_c, msg_c,
         sem_kv, sem_q, sem_sc) = slots[slot]
        base = s * EPT_RAW + ch * BCA
        pltpu.sync_copy(e_hbm.at[pl.ds(eoff + base, BCA)], e_c)
        pltpu.make_async_copy(kv_hbm.at[src_g], kv_rows, sem_kv).wait()
        pltpu.make_async_copy(q_hbm.at[dst_g], q_rows, sem_q).wait()

        @plsc.parallel_loop(0, BCA, unroll=2)
        def _edge(b):
            validv = jnp.broadcast_to(ch * BCA + b, (L,)) < EPT_RAW
            exs = []
            for h in range(4):
                qd = q_rows[b, pl.ds(h * L, L)]
                kh = kv_rows[b, pl.ds(h * L, L)]
                eh = e_c[b, pl.ds(h * L, L)]
                a_h = jnp.sum(qd * (kh + eh))
                av = jnp.broadcast_to(a_h, (L,))
                exb = jnp.where(validv,
                                jnp.exp(jnp.minimum(av * 0.25, 60.0)), 0.0)
                vh = kv_rows[b, pl.ds(64 + h * L, L)]
                msg_c[b, pl.ds(h * L, L)] = exb * (vh + eh)
                exs.append(exb)
            denv = zero16
            for h in range(4):
                denv = jnp.where(iota == h, exs[h], denv)
            msg_c[b, pl.ds(64, L)] = denv

        pltpu.async_copy(msg_c, sh_acc.at[dst_i], sem_sc, add=True)

    _load(0, 0)
    NPAIR = (NCHUNK + 1) // 2

    def _pair(i, _):
        ch0 = 2 * i
        ch1 = ch0 + 1

        @pl.when(jnp.logical_and(ch1 < NCHUNK, i > 0))
        def _():
            _wait_sc(1)

        @pl.when(ch1 < NCHUNK)
        def _():
            _load(ch1, 1)
        _compute(ch0, 0)

        @pl.when(ch0 + 2 < NCHUNK)
        def _():
            _wait_sc(0)
            _load(ch0 + 2, 0)

        @pl.when(ch1 < NCHUNK)
        def _():
            _compute(ch1, 1)
        return 0

    lax.fori_loop(0, NPAIR, _pair, 0)
    _wait_sc(0)
    _wait_sc(1)
    plsc.subcore_barrier()

    r0 = s * ROWS_T
    o0 = c * NP + s * ROWS_T
    pltpu.sync_copy(sh_acc.at[pl.ds(r0, ROWS_T)], acc_hbm.at[pl.ds(o0, ROWS_T)])


@jax.jit
def _sc_attention(q2, kv2, e2, srcp, dstp):
    f32 = jnp.float32
    kern = pl.kernel(
        _attn_body,
        out_type=[jax.ShapeDtypeStruct((NC * NP, AW), f32)],
        mesh=_sc_mesh(),
        scratch_types=(
            [pltpu.VMEM((BCA,), jnp.int32)] * 4
            + [pltpu.VMEM((BCA, 64), f32), pltpu.VMEM((BCA, 128), f32),
               pltpu.VMEM((BCA, 64), f32), pltpu.VMEM((BCA, AW), f32)]
            + [pltpu.VMEM((BCA,), jnp.int32)] * 4
            + [pltpu.VMEM((BCA, 64), f32), pltpu.VMEM((BCA, 128), f32),
               pltpu.VMEM((BCA, AW), f32)]
            + [pltpu.SemaphoreType.DMA] * 6
            + [pltpu.VMEM_SHARED((NP, AW), f32)]
        ),
        **_SC_PARAMS,
    )
    return kern(q2, kv2, e2, srcp, dstp)[0]


def _edge_body(p_hbm, q_hbm, t_hbm, src_hbm, dst_hbm, out_hbm,
               src_i, dst_i, p_rows, q_rows, t_c, o_c, sem1, sem2):
    c = lax.axis_index("c")
    s = lax.axis_index("s")
    w = s * NC + c

    def _chunk(ch, _):
        base = w * EPW_RAW + ch * BC
        pltpu.sync_copy(src_hbm.at[pl.ds(base, BC)], src_i)
        pltpu.sync_copy(dst_hbm.at[pl.ds(base, BC)], dst_i)
        pltpu.sync_copy(t_hbm.at[pl.ds(base, BC)], t_c)
        pltpu.async_copy(p_hbm.at[src_i], p_rows, sem1).wait()
        pltpu.async_copy(q_hbm.at[dst_i], q_rows, sem2).wait()

        def _edge(b, _):
            o_c[b, :] = t_c[b, :] + p_rows[b, :] + q_rows[b, :]
            return 0
        lax.fori_loop(0, BC, _edge, 0)
        pltpu.sync_copy(o_c, out_hbm.at[pl.ds(base, BC)])
        return 0

    lax.fori_loop(0, NCHUNK2, _chunk, 0)


@jax.jit
def _sc_edge(p, q, t, srcp2, dstp2):
    f32 = jnp.float32
    kern = pl.kernel(
        _edge_body,
        out_type=[jax.ShapeDtypeStruct((E + PAD, DE), f32)],
        mesh=_sc_mesh(),
        scratch_types=[
            pltpu.VMEM((BC,), jnp.int32),
            pltpu.VMEM((BC,), jnp.int32),
            pltpu.VMEM((BC, DE), f32),
            pltpu.VMEM((BC, DE), f32),
            pltpu.VMEM((BC, DE), f32),
            pltpu.VMEM((BC, DE), f32),
            pltpu.SemaphoreType.DMA,
            pltpu.SemaphoreType.DMA,
        ],
        **_SC_PARAMS,
    )
    return kern(p, q, t, srcp2, dstp2)[0]


def _erf(x):
    ax = jnp.abs(x)
    t = 1.0 / (1.0 + 0.3275911 * ax)
    poly = t * (0.254829592 + t * (-0.284496736 + t * (1.421413741 +
           t * (-1.453152027 + t * 1.061405429))))
    e = 1.0 - poly * jnp.exp(-ax * ax)
    return jnp.where(x < 0, -e, e)


def _gelu(x):
    return 0.5 * x * (1.0 + _erf(x * np.float32(1.0 / np.sqrt(2.0))))


def _stats_rows(xb):
    s0 = jnp.sum(xb, axis=0, keepdims=True)
    s1 = jnp.sum(xb * xb, axis=0, keepdims=True)
    return jnp.concatenate([s0, s1, jnp.zeros((6, xb.shape[1]), jnp.float32)], 0)


def _kpre_body(na, wall, ball, q_o, kv_o, skip_o):
    x = na[...]
    qkvs = x @ wall[...] + ball[...]
    q = qkvs[:, :128]
    k = qkvs[:, 128:256]
    v = qkvs[:, 256:384]
    skip_o[...] = qkvs[:, 384:512]
    q_o[0] = q[:, :64]
    q_o[1] = q[:, 64:]
    kv_o[0] = jnp.concatenate([k[:, :64], v[:, :64]], axis=1)
    kv_o[1] = jnp.concatenate([k[:, 64:], v[:, 64:]], axis=1)


def k_pre(na, wall, ball):
    f32 = jnp.float32
    grid = N // BN_N
    return pl.pallas_call(
        _kpre_body,
        grid=(grid,),
        in_specs=[
            pl.BlockSpec((BN_N, 128), lambda i: (i, 0)),
            pl.BlockSpec((128, 512), lambda i: (0, 0)),
            pl.BlockSpec((1, 512), lambda i: (0, 0)),
        ],
        out_specs=[
            pl.BlockSpec((2, BN_N, 64), lambda i: (0, i, 0)),
            pl.BlockSpec((2, BN_N, 128), lambda i: (0, i, 0)),
            pl.BlockSpec((BN_N, 128), lambda i: (i, 0)),
        ],
        out_shape=[
            jax.ShapeDtypeStruct((2, N, 64), f32),
            jax.ShapeDtypeStruct((2, N, 128), f32),
            jax.ShapeDtypeStruct((N, 128), f32),
        ],
    )(na, wall, ball)


def _ke_body(ea, we, e_o):
    e = ea[...] @ we[...]
    e_o[0] = e[:, :64]
    e_o[1] = e[:, 64:]


def k_e(ea, we):
    f32 = jnp.float32
    grid = E // BN_E + 1
    return pl.pallas_call(
        _ke_body,
        grid=(grid,),
        in_specs=[
            pl.BlockSpec((BN_E, 16), lambda i: (jnp.minimum(i, E // BN_E - 1), 0)),
            pl.BlockSpec((16, 128), lambda i: (0, 0)),
        ],
        out_specs=[pl.BlockSpec((2, BN_E, 64), lambda i: (0, i, 0))],
        out_shape=[jax.ShapeDtypeStruct((2, ES, 64), f32)],
    )(ea, we)[0]


def _k1_body(outv, denr, skipx, x_o, st_o):
    i = pl.program_id(0)
    x = outv[...] / (denr[...] + 1e-16) + skipx[...]
    x_o[...] = x
    rows = _stats_rows(x)

    @pl.when(i == 0)
    def _():
        st_o[...] = rows

    @pl.when(i > 0)
    def _():
        st_o[...] = st_o[...] + rows


def k1(outv, denr, skipx):
    f32 = jnp.float32
    grid = N // BN_N
    return pl.pallas_call(
        _k1_body,
        grid=(grid,),
        in_specs=[
            pl.BlockSpec((BN_N, 128), lambda i: (i, 0)),
            pl.BlockSpec((BN_N, 128), lambda i: (i, 0)),
            pl.BlockSpec((BN_N, 128), lambda i: (i, 0)),
        ],
        out_specs=[
            pl.BlockSpec((BN_N, 128), lambda i: (i, 0)),
            pl.BlockSpec((8, 128), lambda i: (0, 0)),
        ],
        out_shape=[
            jax.ShapeDtypeStruct((N, 128), f32),
            jax.ShapeDtypeStruct((8, 128), f32),
        ],
    )(outv, denr, skipx)


def _k2_body(x, st1, g1, b1n, w1, b1f, w2, b2f, z_o, st_o):
    i = pl.program_id(0)
    mean = st1[0:1, :] * np.float32(1.0 / N)
    var = st1[1:2, :] * np.float32(1.0 / N) - mean * mean
    y1 = (x[...] - mean) * lax.rsqrt(var + 1e-5) * g1[...] + b1n[...]
    h = _gelu(y1 @ w1[...] + b1f[...])
    z = h @ w2[...] + b2f[...] + y1
    z_o[...] = z
    rows = _stats_rows(z)

    @pl.when(i == 0)
    def _():
        st_o[...] = rows

    @pl.when(i > 0)
    def _():
        st_o[...] = st_o[...] + rows


def k2(x, st1, g1, b1n, w1, b1f, w2, b2f):
    f32 = jnp.float32
    grid = N // BN_N
    return pl.pallas_call(
        _k2_body,
        grid=(grid,),
        in_specs=[
            pl.BlockSpec((BN_N, 128), lambda i: (i, 0)),
            pl.BlockSpec((8, 128), lambda i: (0, 0)),
            pl.BlockSpec((1, 128), lambda i: (0, 0)),
            pl.BlockSpec((1, 128), lambda i: (0, 0)),
            pl.BlockSpec((128, 1024), lambda i: (0, 0)),
            pl.BlockSpec((1, 1024), lambda i: (0, 0)),
            pl.BlockSpec((1024, 128), lambda i: (0, 0)),
            pl.BlockSpec((1, 128), lambda i: (0, 0)),
        ],
        out_specs=[
            pl.BlockSpec((BN_N, 128), lambda i: (i, 0)),
            pl.BlockSpec((8, 128), lambda i: (0, 0)),
        ],
        out_shape=[
            jax.ShapeDtypeStruct((N, 128), f32),
            jax.ShapeDtypeStruct((8, 128), f32),
        ],
    )(x, st1, g1, b1n, w1, b1f, w2, b2f)


def _k3_body(z, st2, g2, b2n, wpq, no_o, pq_o):
    mean = st2[0:1, :] * np.float32(1.0 / N)
    var = st2[1:2, :] * np.float32(1.0 / N) - mean * mean
    y2 = (z[...] - mean) * lax.rsqrt(var + 1e-5) * g2[...] + b2n[...]
    no_o[...] = _gelu(y2)
    pq_o[...] = y2 @ wpq[...]


def k3(z, st2, g2, b2n, wpq):
    f32 = jnp.float32
    grid = N // BN_N
    return pl.pallas_call(
        _k3_body,
        grid=(grid,),
        in_specs=[
            pl.BlockSpec((BN_N, 128), lambda i: (i, 0)),
            pl.BlockSpec((8, 128), lambda i: (0, 0)),
            pl.BlockSpec((1, 128), lambda i: (0, 0)),
            pl.BlockSpec((1, 128), lambda i: (0, 0)),
            pl.BlockSpec((128, 32), lambda i: (0, 0)),
        ],
        out_specs=[
            pl.BlockSpec((BN_N, 128), lambda i: (i, 0)),
            pl.BlockSpec((BN_N, 32), lambda i: (i, 0)),
        ],
        out_shape=[
            jax.ShapeDtypeStruct((N, 128), f32),
            jax.ShapeDtypeStruct((N, 32), f32),
        ],
    )(z, st2, g2, b2n, wpq)


def _kt_body(ea, wea, bep1, t_o):
    t_o[...] = ea[...] @ wea[...] + bep1[...]


def k_t(eap, weabd, bep1t):
    f32 = jnp.float32
    grid = EP8 // BN_P
    return pl.pallas_call(
        _kt_body,
        grid=(grid,),
        in_specs=[
            pl.BlockSpec((BN_P, 128), lambda i: (i, 0)),
            pl.BlockSpec((128, 128), lambda i: (0, 0)),
            pl.BlockSpec((1, 128), lambda i: (0, 0)),
        ],
        out_specs=[pl.BlockSpec((BN_P, 128), lambda i: (i, 0))],
        out_shape=[jax.ShapeDtypeStruct((EP8, 128), f32)],
    )(eap, weabd, bep1t)[0]


def _k5a_body(eop, wep2, bep2, u_o, st_o):
    i = pl.program_id(0)
    u = _gelu(eop[...]) @ wep2[...] + bep2[...]
    u_o[...] = u
    rows = _stats_rows(u)

    @pl.when(i == 0)
    def _():
        st_o[...] = rows

    @pl.when(i > 0)
    def _():
        st_o[...] = st_o[...] + rows


def k5a(eop, wep2bd, bep2t):
    f32 = jnp.float32
    grid = EP8 // BN_P
    return pl.pallas_call(
        _k5a_body,
        grid=(grid,),
        in_specs=[
            pl.BlockSpec((BN_P, 128), lambda i: (i, 0)),
            pl.BlockSpec((128, 128), lambda i: (0, 0)),
            pl.BlockSpec((1, 128), lambda i: (0, 0)),
        ],
        out_specs=[
            pl.BlockSpec((BN_P, 128), lambda i: (i, 0)),
            pl.BlockSpec((8, 128), lambda i: (0, 0)),
        ],
        out_shape=[
            jax.ShapeDtypeStruct((EP8, 128), f32),
            jax.ShapeDtypeStruct((8, 128), f32),
        ],
    )(eop, wep2bd, bep2t)


def _k5b_body(u, mt, vt, ge, be, eo_o):
    y = (u[...] - mt[...]) * lax.rsqrt(vt[...] + 1e-5) * ge[...] + be[...]
    eo_o[...] = _gelu(y)


def k5b(u, mt, vt, ge, be):
    f32 = jnp.float32
    grid = EP8 // BN_P
    return pl.pallas_call(
        _k5b_body,
        grid=(grid,),
        in_specs=[
            pl.BlockSpec((BN_P, 128), lambda i: (i, 0)),
            pl.BlockSpec((1, 128), lambda i: (0, 0)),
            pl.BlockSpec((1, 128), lambda i: (0, 0)),
            pl.BlockSpec((1, 128), lambda i: (0, 0)),
            pl.BlockSpec((1, 128), lambda i: (0, 0)),
        ],
        out_specs=[pl.BlockSpec((BN_P, 128), lambda i: (i, 0))],
        out_shape=[jax.ShapeDtypeStruct((EP8, 128), f32)],
    )(u, mt, vt, ge, be)[0]


def kernel(node_attr, edge_attr, Wq, bq, Wk, bk, Wv, bv, We, Wskip, bskip,
           gamma1, beta1, gamma2, beta2, W1, b1, W2, b2, Wep1, bep1, Wep2,
           bep2, gammaE, betaE, edge_index, batch):
    src = edge_index[0]
    dst = edge_index[1]

    A = Wep1[DE:DE + D]
    B = Wep1[DE + D:DE + 2 * D]
    Wpq = jnp.concatenate([A + B, B - A], axis=1)
    Wea = Wep1[:DE]

    Wall = jnp.concatenate([Wq, Wk, Wv, Wskip], axis=1)
    ball = jnp.concatenate([bq, bk, bv, bskip]).reshape(1, 512)
    q3, kv3, skipx = k_pre(node_attr, Wall, ball)
    q2 = q3.reshape(NC * N, 64)
    kv2 = kv3.reshape(NC * N, 128)
    e2 = k_e(edge_attr, We).reshape(NC * ES, 64)

    srcp = jnp.pad(src, (0, PAD))
    dstp = jnp.pad(dst, (0, PAD))

    acc = _sc_attention(q2, kv2, e2, srcp, dstp)
    outv = jnp.concatenate([acc[:N, :64], acc[NP:NP + N, :64]], axis=1)
    den8 = jnp.concatenate([acc[:N, 64:68], acc[NP:NP + N, 64:68]], axis=1)
    den_rep = jnp.repeat(den8, C, axis=1)

    x, st1 = k1(outv, den_rep, skipx)
    z, st2 = k2(x, st1, gamma1.reshape(1, 128), beta1.reshape(1, 128),
                W1, b1.reshape(1, DFF), W2, b2.reshape(1, 128))
    node_out, PQ = k3(z, st2, gamma2.reshape(1, 128), beta2.reshape(1, 128),
                      Wpq)

    P = PQ[:, :DE]
    Qm = PQ[:, DE:]
    eap8 = edge_attr.reshape(EP8, 128)
    Weabd = jax.scipy.linalg.block_diag(*([Wea] * 8))
    bep1t = jnp.tile(bep1, 8).reshape(1, 128)
    T = jnp.pad(k_t(eap8, Weabd, bep1t).reshape(E, DE), ((0, PAD), (0, 0)))

    eo_pre = _sc_edge(P, Qm, T, srcp, dstp)[:E].reshape(EP8, 128)
    Wep2bd = jax.scipy.linalg.block_diag(*([Wep2] * 8))
    bep2t = jnp.tile(bep2, 8).reshape(1, 128)
    u, stE = k5a(eo_pre, Wep2bd, bep2t)
    msum = jnp.sum(stE[0].reshape(8, DE), axis=0) * np.float32(1.0 / E)
    ssum = jnp.sum(stE[1].reshape(8, DE), axis=0) * np.float32(1.0 / E)
    var = ssum - msum * msum
    mt = jnp.tile(msum, 8).reshape(1, 128)
    vt = jnp.tile(var, 8).reshape(1, 128)
    get = jnp.tile(gammaE, 8).reshape(1, 128)
    bet = jnp.tile(betaE, 8).reshape(1, 128)
    eo = k5b(u, mt, vt, get, bet).reshape(E, DE)
    return (node_out, eo)


# speedup vs baseline: 25.8315x; 1.2774x over previous
import functools

import jax
import jax.numpy as jnp
import numpy as np
from jax import lax
from jax.experimental import pallas as pl
from jax.experimental.pallas import tpu as pltpu
from jax.experimental.pallas import tpu_sc as plsc

N = 10000
E = 320000
D = 128
H = 8
C = 16
DE = 16
DFF = 1024

NC = 2
NS = 16
L = 16
PAD = 128

EPT_RAW = E // NS
BCA = 112
NCHUNK = (EPT_RAW + BCA - 1) // BCA
NP = 10112
ROWS_T = NP // NS
AW = 80

EPW_RAW = E // (NC * NS)
BC = 128
NCHUNK2 = (EPW_RAW + BC - 1) // BC
KE_B = 1000
ES2 = 41 * 8 * KE_B
TR = 44000
NPACK = EPW_RAW // 8

BN_N = 1000
BN_E = 2560
EP8 = E // 8
BN_P = 4000


def _iota16():
    return lax.broadcasted_iota(jnp.int32, (L,), 0)


def _sc_mesh():
    return plsc.VectorSubcoreMesh(
        core_axis_name="c", subcore_axis_name="s", num_cores=NC,
        num_subcores=NS)


_SC_PARAMS = dict(
    compiler_params=pltpu.CompilerParams(
        needs_layout_passes=False, use_tc_tiling_on_sc=False))


def _attn_body(q_hbm, kv_hbm, e_hbm, src_hbm, dst_hbm,
               acc_hbm,
               src_i0, dst_i0, src_g0, dst_g0, q_rows0, kv_rows0, e_c0,
               msg_c0,
               src_i1, dst_i1, src_g1, dst_g1, q_rows1, kv_rows1,
               msg_c1,
               sem_kv0, sem_q0, sem_sc0, sem_kv1, sem_q1, sem_sc1,
               sh_acc):
    c = lax.axis_index("c")
    s = lax.axis_index("s")
    zero16 = jnp.zeros((L,), jnp.float32)
    iota = _iota16()

    def _z(i, _):
        for jj in range(AW // L):
            msg_c0[i, pl.ds(jj * L, L)] = zero16
        return 0
    lax.fori_loop(0, BCA, _z, 0)

    NZ = (ROWS_T + BCA - 1) // BCA
    for t in range(NZ):
        r0 = s * ROWS_T + t * BCA
        nr = BCA if t < NZ - 1 else ROWS_T - (NZ - 1) * BCA
        pltpu.sync_copy(msg_c0.at[pl.ds(0, nr)], sh_acc.at[pl.ds(r0, nr)])
    plsc.subcore_barrier()

    coff = c * N
    coff64 = c * 64

    slots = (
        (src_i0, dst_i0, src_g0, dst_g0, q_rows0, kv_rows0, e_c0, msg_c0,
         sem_kv0, sem_q0, sem_sc0),
        (src_i1, dst_i1, src_g1, dst_g1, q_rows1, kv_rows1, e_c0, msg_c1,
         sem_kv1, sem_q1, sem_sc1),
    )

    def _load(ch, slot):
        (src_i, dst_i, src_g, dst_g, q_rows, kv_rows, e_c, msg_c,
         sem_kv, sem_q, sem_sc) = slots[slot]
        base = s * EPT_RAW + ch * BCA
        pltpu.sync_copy(src_hbm.at[pl.ds(base, BCA)], src_i)
        pltpu.sync_copy(dst_hbm.at[pl.ds(base, BCA)], dst_i)
        for g in range(BCA // L):
            sl = pl.ds(g * L, L)
            src_g[sl] = src_i[sl] + coff
            dst_g[sl] = dst_i[sl] + coff
        pltpu.async_copy(kv_hbm.at[src_g], kv_rows, sem_kv)
        pltpu.async_copy(q_hbm.at[dst_g], q_rows, sem_q)

    def _wait_sc(slot):
        (src_i, dst_i, src_g, dst_g, q_rows, kv_rows, e_c, msg_c,
         sem_kv, sem_q, sem_sc) = slots[slot]
        pltpu.make_async_copy(msg_c, sh_acc.at[dst_i], sem_sc).wait()

    def _compute(ch, slot):
        (src_i, dst_i, src_g, dst_g, q_rows, kv_rows, e_c, msg_c,
         sem_kv, sem_q, sem_sc) = slots[slot]
        base = s * EPT_RAW + ch * BCA
        pltpu.sync_copy(e_hbm.at[pl.ds(base, BCA)], e_c)
        pltpu.make_async_copy(kv_hbm.at[src_g], kv_rows, sem_kv).wait()
        pltpu.make_async_copy(q_hbm.at[dst_g], q_rows, sem_q).wait()

        @plsc.parallel_loop(0, BCA, unroll=2)
        def _edge(b):
            validv = jnp.broadcast_to(ch * BCA + b, (L,)) < EPT_RAW
            exs = []
            for h in range(4):
                qd = q_rows[b, pl.ds(h * L, L)]
                kh = kv_rows[b, pl.ds(h * L, L)]
                eh = e_c[b, pl.ds(coff64 + h * L, L)]
                a_h = jnp.sum(qd * (kh + eh))
                av = jnp.broadcast_to(a_h, (L,))
                exb = jnp.where(validv,
                                jnp.exp(jnp.minimum(av * 0.25, 60.0)), 0.0)
                vh = kv_rows[b, pl.ds(64 + h * L, L)]
                msg_c[b, pl.ds(h * L, L)] = exb * (vh + eh)
                exs.append(exb)
            denv = zero16
            for h in range(4):
                denv = jnp.where(iota == h, exs[h], denv)
            msg_c[b, pl.ds(64, L)] = denv

        pltpu.async_copy(msg_c, sh_acc.at[dst_i], sem_sc, add=True)

    _load(0, 0)
    NPAIR = (NCHUNK + 1) // 2

    def _pair(i, _):
        ch0 = 2 * i
        ch1 = ch0 + 1

        @pl.when(jnp.logical_and(ch1 < NCHUNK, i > 0))
        def _():
            _wait_sc(1)

        @pl.when(ch1 < NCHUNK)
        def _():
            _load(ch1, 1)
        _compute(ch0, 0)

        @pl.when(ch0 + 2 < NCHUNK)
        def _():
            _wait_sc(0)
            _load(ch0 + 2, 0)

        @pl.when(ch1 < NCHUNK)
        def _():
            _compute(ch1, 1)
        return 0

    lax.fori_loop(0, NPAIR, _pair, 0)
    _wait_sc(0)
    _wait_sc(1)
    plsc.subcore_barrier()

    r0 = s * ROWS_T
    o0 = c * NP + s * ROWS_T
    pltpu.sync_copy(sh_acc.at[pl.ds(r0, ROWS_T)], acc_hbm.at[pl.ds(o0, ROWS_T)])


@jax.jit
def _sc_attention(q2, kv2, e2, srcp, dstp):
    f32 = jnp.float32
    kern = pl.kernel(
        _attn_body,
        out_type=[jax.ShapeDtypeStruct((NC * NP, AW), f32)],
        mesh=_sc_mesh(),
        scratch_types=(
            [pltpu.VMEM((BCA,), jnp.int32)] * 4
            + [pltpu.VMEM((BCA, 64), f32), pltpu.VMEM((BCA, 128), f32),
               pltpu.VMEM((BCA, 128), f32), pltpu.VMEM((BCA, AW), f32)]
            + [pltpu.VMEM((BCA,), jnp.int32)] * 4
            + [pltpu.VMEM((BCA, 64), f32), pltpu.VMEM((BCA, 128), f32),
               pltpu.VMEM((BCA, AW), f32)]
            + [pltpu.SemaphoreType.DMA] * 6
            + [pltpu.VMEM_SHARED((NP, AW), f32)]
        ),
        **_SC_PARAMS,
    )
    return kern(q2, kv2, e2, srcp, dstp)[0]


def _edge_body(p_hbm, q_hbm, t_hbm, src_hbm, dst_hbm, out_hbm,
               src_i, dst_i, p_rows, q_rows, t_c, o_c, sem1, sem2):
    c = lax.axis_index("c")
    s = lax.axis_index("s")
    w = s * NC + c

    def _chunk(ch, _):
        base = w * EPW_RAW + ch * BC
        pbase = w * NPACK + ch * (BC // 8)
        pltpu.sync_copy(src_hbm.at[pl.ds(base, BC)], src_i)
        pltpu.sync_copy(dst_hbm.at[pl.ds(base, BC)], dst_i)
        pltpu.sync_copy(t_hbm.at[pl.ds(pbase, BC // 8)], t_c)
        pltpu.async_copy(p_hbm.at[src_i], p_rows, sem1).wait()
        pltpu.async_copy(q_hbm.at[dst_i], q_rows, sem2).wait()

        @plsc.parallel_loop(0, BC, unroll=2)
        def _edge(b):
            pr = b // 8
            lo = (b % 8) * DE
            o_c[pr, pl.ds(lo, DE)] = (t_c[pr, pl.ds(lo, DE)]
                                      + p_rows[b, :] + q_rows[b, :])
        pltpu.sync_copy(o_c, out_hbm.at[pl.ds(pbase, BC // 8)])
        return 0

    lax.fori_loop(0, NCHUNK2, _chunk, 0)


@jax.jit
def _sc_edge(p, q, t, srcp2, dstp2):
    f32 = jnp.float32
    kern = pl.kernel(
        _edge_body,
        out_type=[jax.ShapeDtypeStruct((TR, 128), f32)],
        mesh=_sc_mesh(),
        scratch_types=[
            pltpu.VMEM((BC,), jnp.int32),
            pltpu.VMEM((BC,), jnp.int32),
            pltpu.VMEM((BC, DE), f32),
            pltpu.VMEM((BC, DE), f32),
            pltpu.VMEM((BC // 8, 128), f32),
            pltpu.VMEM((BC // 8, 128), f32),
            pltpu.SemaphoreType.DMA,
            pltpu.SemaphoreType.DMA,
        ],
        **_SC_PARAMS,
    )
    return kern(p, q, t, srcp2, dstp2)[0]


def _erf(x):
    ax = jnp.abs(x)
    t = 1.0 / (1.0 + 0.3275911 * ax)
    poly = t * (0.254829592 + t * (-0.284496736 + t * (1.421413741 +
           t * (-1.453152027 + t * 1.061405429))))
    e = 1.0 - poly * jnp.exp(-ax * ax)
    return jnp.where(x < 0, -e, e)


def _gelu(x):
    return 0.5 * x * (1.0 + _erf(x * np.float32(1.0 / np.sqrt(2.0))))


def _stats_rows(xb):
    s0 = jnp.sum(xb, axis=0, keepdims=True)
    s1 = jnp.sum(xb * xb, axis=0, keepdims=True)
    return jnp.concatenate([s0, s1, jnp.zeros((6, xb.shape[1]), jnp.float32)], 0)


def _kpre_body(na, wall, ball, q_o, kv_o, skip_o):
    cc = pl.program_id(0)
    x = na[...]
    qkvs = x @ wall[...] + ball[...]
    q = qkvs[:, :128]
    k = qkvs[:, 128:256]
    v = qkvs[:, 256:384]
    skip_o[...] = qkvs[:, 384:512]
    q_o[...] = jnp.where(cc == 0, q[:, :64], q[:, 64:])
    kv_o[...] = jnp.where(
        cc == 0,
        jnp.concatenate([k[:, :64], v[:, :64]], axis=1),
        jnp.concatenate([k[:, 64:], v[:, 64:]], axis=1))


def k_pre(na, wall, ball):
    f32 = jnp.float32
    grid = N // BN_N
    return pl.pallas_call(
        _kpre_body,
        grid=(2, grid),
        in_specs=[
            pl.BlockSpec((BN_N, 128), lambda c, i: (i, 0)),
            pl.BlockSpec((128, 512), lambda c, i: (0, 0)),
            pl.BlockSpec((1, 512), lambda c, i: (0, 0)),
        ],
        out_specs=[
            pl.BlockSpec((BN_N, 64), lambda c, i: (c * (N // BN_N) + i, 0)),
            pl.BlockSpec((BN_N, 128), lambda c, i: (c * (N // BN_N) + i, 0)),
            pl.BlockSpec((BN_N, 128), lambda c, i: (i, 0)),
        ],
        out_shape=[
            jax.ShapeDtypeStruct((NC * N, 64), f32),
            jax.ShapeDtypeStruct((NC * N, 128), f32),
            jax.ShapeDtypeStruct((N, 128), f32),
        ],
    )(na, wall, ball)


def _ke_body(eap, we, e_o):
    x = eap[...]
    parts = [x[:, ss * 16:(ss + 1) * 16] @ we[...] for ss in range(8)]
    e_o[...] = jnp.stack(parts, axis=1).reshape(KE_B * 8, 128)


def k_e(eap8, we):
    f32 = jnp.float32
    nb = EP8 // KE_B
    return pl.pallas_call(
        _ke_body,
        grid=(nb + 1,),
        in_specs=[
            pl.BlockSpec((KE_B, 128), lambda i: (jnp.minimum(i, nb - 1), 0)),
            pl.BlockSpec((16, 128), lambda i: (0, 0)),
        ],
        out_specs=[pl.BlockSpec((KE_B * 8, 128), lambda i: (i, 0))],
        out_shape=[jax.ShapeDtypeStruct((ES2, 128), f32)],
    )(eap8, we)[0]


def _k1_body(outv, denr, skipx, x_o, st_o):
    i = pl.program_id(0)
    x = outv[...] / (denr[...] + 1e-16) + skipx[...]
    x_o[...] = x
    rows = _stats_rows(x)

    @pl.when(i == 0)
    def _():
        st_o[...] = rows

    @pl.when(i > 0)
    def _():
        st_o[...] = st_o[...] + rows


def k1(outv, denr, skipx):
    f32 = jnp.float32
    grid = N // BN_N
    return pl.pallas_call(
        _k1_body,
        grid=(grid,),
        in_specs=[
            pl.BlockSpec((BN_N, 128), lambda i: (i, 0)),
            pl.BlockSpec((BN_N, 128), lambda i: (i, 0)),
            pl.BlockSpec((BN_N, 128), lambda i: (i, 0)),
        ],
        out_specs=[
            pl.BlockSpec((BN_N, 128), lambda i: (i, 0)),
            pl.BlockSpec((8, 128), lambda i: (0, 0)),
        ],
        out_shape=[
            jax.ShapeDtypeStruct((N, 128), f32),
            jax.ShapeDtypeStruct((8, 128), f32),
        ],
    )(outv, denr, skipx)


def _k2_body(x, st1, g1, b1n, w1, b1f, w2, b2f, z_o, st_o):
    i = pl.program_id(0)
    mean = st1[0:1, :] * np.float32(1.0 / N)
    var = st1[1:2, :] * np.float32(1.0 / N) - mean * mean
    y1 = (x[...] - mean) * lax.rsqrt(var + 1e-5) * g1[...] + b1n[...]
    h = _gelu(y1 @ w1[...] + b1f[...])
    z = h @ w2[...] + b2f[...] + y1
    z_o[...] = z
    rows = _stats_rows(z)

    @pl.when(i == 0)
    def _():
        st_o[...] = rows

    @pl.when(i > 0)
    def _():
        st_o[...] = st_o[...] + rows


def k2(x, st1, g1, b1n, w1, b1f, w2, b2f):
    f32 = jnp.float32
    grid = N // BN_N
    return pl.pallas_call(
        _k2_body,
        grid=(grid,),
        in_specs=[
            pl.BlockSpec((BN_N, 128), lambda i: (i, 0)),
            pl.BlockSpec((8, 128), lambda i: (0, 0)),
            pl.BlockSpec((1, 128), lambda i: (0, 0)),
            pl.BlockSpec((1, 128), lambda i: (0, 0)),
            pl.BlockSpec((128, 1024), lambda i: (0, 0)),
            pl.BlockSpec((1, 1024), lambda i: (0, 0)),
            pl.BlockSpec((1024, 128), lambda i: (0, 0)),
            pl.BlockSpec((1, 128), lambda i: (0, 0)),
        ],
        out_specs=[
            pl.BlockSpec((BN_N, 128), lambda i: (i, 0)),
            pl.BlockSpec((8, 128), lambda i: (0, 0)),
        ],
        out_shape=[
            jax.ShapeDtypeStruct((N, 128), f32),
            jax.ShapeDtypeStruct((8, 128), f32),
        ],
    )(x, st1, g1, b1n, w1, b1f, w2, b2f)


def _k3_body(z, st2, g2, b2n, wpq, no_o, pq_o):
    mean = st2[0:1, :] * np.float32(1.0 / N)
    var = st2[1:2, :] * np.float32(1.0 / N) - mean * mean
    y2 = (z[...] - mean) * lax.rsqrt(var + 1e-5) * g2[...] + b2n[...]
    no_o[...] = _gelu(y2)
    pq_o[...] = y2 @ wpq[...]


def k3(z, st2, g2, b2n, wpq):
    f32 = jnp.float32
    grid = N // BN_N
    return pl.pallas_call(
        _k3_body,
        grid=(grid,),
        in_specs=[
            pl.BlockSpec((BN_N, 128), lambda i: (i, 0)),
            pl.BlockSpec((8, 128), lambda i: (0, 0)),
            pl.BlockSpec((1, 128), lambda i: (0, 0)),
            pl.BlockSpec((1, 128), lambda i: (0, 0)),
            pl.BlockSpec((128, 32), lambda i: (0, 0)),
        ],
        out_specs=[
            pl.BlockSpec((BN_N, 128), lambda i: (i, 0)),
            pl.BlockSpec((BN_N, 32), lambda i: (i, 0)),
        ],
        out_shape=[
            jax.ShapeDtypeStruct((N, 128), f32),
            jax.ShapeDtypeStruct((N, 32), f32),
        ],
    )(z, st2, g2, b2n, wpq)


def _kt_body(ea, wea, bep1, t_o):
    t_o[...] = ea[...] @ wea[...] + bep1[...]


def k_t(eap, weabd, bep1t):
    f32 = jnp.float32
    grid = TR // BN_P
    nb = EP8 // BN_P
    return pl.pallas_call(
        _kt_body,
        grid=(grid,),
        in_specs=[
            pl.BlockSpec((BN_P, 128), lambda i: (jnp.minimum(i, nb - 1), 0)),
            pl.BlockSpec((128, 128), lambda i: (0, 0)),
            pl.BlockSpec((1, 128), lambda i: (0, 0)),
        ],
        out_specs=[pl.BlockSpec((BN_P, 128), lambda i: (i, 0))],
        out_shape=[jax.ShapeDtypeStruct((TR, 128), f32)],
    )(eap, weabd, bep1t)[0]


def _k5a_body(eop, wep2, bep2, u_o, st_o):
    i = pl.program_id(0)
    u = _gelu(eop[...]) @ wep2[...] + bep2[...]
    u_o[...] = u
    rows = _stats_rows(u)

    @pl.when(i == 0)
    def _():
        st_o[...] = rows

    @pl.when(i > 0)
    def _():
        st_o[...] = st_o[...] + rows


def k5a(eop, wep2bd, bep2t):
    f32 = jnp.float32
    grid = EP8 // BN_P
    return pl.pallas_call(
        _k5a_body,
        grid=(grid,),
        in_specs=[
            pl.BlockSpec((BN_P, 128), lambda i: (i, 0)),
            pl.BlockSpec((128, 128), lambda i: (0, 0)),
            pl.BlockSpec((1, 128), lambda i: (0, 0)),
        ],
        out_specs=[
            pl.BlockSpec((BN_P, 128), lambda i: (i, 0)),
            pl.BlockSpec((8, 128), lambda i: (0, 0)),
        ],
        out_shape=[
            jax.ShapeDtypeStruct((EP8, 128), f32),
            jax.ShapeDtypeStruct((8, 128), f32),
        ],
    )(eop, wep2bd, bep2t)


def _k5b_body(u, mt, vt, ge, be, eo_o):
    y = (u[...] - mt[...]) * lax.rsqrt(vt[...] + 1e-5) * ge[...] + be[...]
    eo_o[...] = _gelu(y)


def k5b(u, mt, vt, ge, be):
    f32 = jnp.float32
    grid = EP8 // BN_P
    return pl.pallas_call(
        _k5b_body,
        grid=(grid,),
        in_specs=[
            pl.BlockSpec((BN_P, 128), lambda i: (i, 0)),
            pl.BlockSpec((1, 128), lambda i: (0, 0)),
            pl.BlockSpec((1, 128), lambda i: (0, 0)),
            pl.BlockSpec((1, 128), lambda i: (0, 0)),
            pl.BlockSpec((1, 128), lambda i: (0, 0)),
        ],
        out_specs=[pl.BlockSpec((BN_P, 128), lambda i: (i, 0))],
        out_shape=[jax.ShapeDtypeStruct((EP8, 128), f32)],
    )(u, mt, vt, ge, be)[0]


def kernel(node_attr, edge_attr, Wq, bq, Wk, bk, Wv, bv, We, Wskip, bskip,
           gamma1, beta1, gamma2, beta2, W1, b1, W2, b2, Wep1, bep1, Wep2,
           bep2, gammaE, betaE, edge_index, batch):
    src = edge_index[0]
    dst = edge_index[1]

    A = Wep1[DE:DE + D]
    B = Wep1[DE + D:DE + 2 * D]
    Wpq = jnp.concatenate([A + B, B - A], axis=1)
    Wea = Wep1[:DE]

    Wall = jnp.concatenate([Wq, Wk, Wv, Wskip], axis=1)
    ball = jnp.concatenate([bq, bk, bv, bskip]).reshape(1, 512)
    q2, kv2, skipx = k_pre(node_attr, Wall, ball)
    eap8 = edge_attr.reshape(EP8, 128)
    e2 = k_e(eap8, We)

    srcp = jnp.pad(src, (0, PAD))
    dstp = jnp.pad(dst, (0, PAD))

    acc = _sc_attention(q2, kv2, e2, srcp, dstp)
    outv = jnp.concatenate([acc[:N, :64], acc[NP:NP + N, :64]], axis=1)
    den8 = jnp.concatenate([acc[:N, 64:68], acc[NP:NP + N, 64:68]], axis=1)
    den_rep = jnp.repeat(den8, C, axis=1)

    x, st1 = k1(outv, den_rep, skipx)
    z, st2 = k2(x, st1, gamma1.reshape(1, 128), beta1.reshape(1, 128),
                W1, b1.reshape(1, DFF), W2, b2.reshape(1, 128))
    node_out, PQ = k3(z, st2, gamma2.reshape(1, 128), beta2.reshape(1, 128),
                      Wpq)

    P = PQ[:, :DE]
    Qm = PQ[:, DE:]
    Weabd = jax.scipy.linalg.block_diag(*([Wea] * 8))
    bep1t = jnp.tile(bep1, 8).reshape(1, 128)
    T = k_t(eap8, Weabd, bep1t)

    eo_pre = _sc_edge(P, Qm, T, srcp, dstp)
    Wep2bd = jax.scipy.linalg.block_diag(*([Wep2] * 8))
    bep2t = jnp.tile(bep2, 8).reshape(1, 128)
    u, stE = k5a(eo_pre, Wep2bd, bep2t)
    msum = jnp.sum(stE[0].reshape(8, DE), axis=0) * np.float32(1.0 / E)
    ssum = jnp.sum(stE[1].reshape(8, DE), axis=0) * np.float32(1.0 / E)
    var = ssum - msum * msum
    mt = jnp.tile(msum, 8).reshape(1, 128)
    vt = jnp.tile(var, 8).reshape(1, 128)
    get = jnp.tile(gammaE, 8).reshape(1, 128)
    bet = jnp.tile(betaE, 8).reshape(1, 128)
    eo = k5b(u, mt, vt, get, bet).reshape(E, DE)
    return (node_out, eo)
